# baseline (speedup 1.0000x reference)
import functools

import jax
import jax.numpy as jnp
from jax import lax
from jax.experimental import pallas as pl
from jax.experimental.pallas import tpu as pltpu
from jax.experimental.pallas import tpu_sc as plsc

N = 10000
E = 160000
D = 256
DE = 16
DG = 128
H = 512
DOUT = 256

_BF = jnp.bfloat16


_PROJ_BN = 1000


def _proj_kernel(x_ref, w_ref, o_ref):
    acc = jnp.dot(
        x_ref[...].astype(_BF), w_ref[...].astype(_BF),
        preferred_element_type=jnp.float32)
    bits = jax.lax.bitcast_convert_type(
        acc.astype(_BF).astype(jnp.float32), jnp.int32) >> 16
    lo = bits[:, :H // 2] & jnp.int32(0xFFFF)
    hi = bits[:, H // 2:] << 16
    o_ref[...] = hi | lo


def _project_nodes(x, w_sr):
    nb = N // _PROJ_BN
    return pl.pallas_call(
        _proj_kernel,
        grid=(2, nb),
        in_specs=[
            pl.BlockSpec((_PROJ_BN, D), lambda j, i: (i, 0)),
            pl.BlockSpec((D, H), lambda j, i: (0, j)),
        ],
        out_specs=pl.BlockSpec((_PROJ_BN, H // 2), lambda j, i: (j * nb + i, 0)),
        out_shape=jax.ShapeDtypeStruct((2 * N, H // 2), jnp.int32),
    )(x, w_sr)


_NC = 2
_NS = 16
_NW = _NC * _NS
_NSLICE = 2
_E_S = E // _NSLICE
_B_SLICE = 2 * _E_S
_PER_W = _B_SLICE // _NW
_CHUNK = 40
_NCHUNK = _PER_W // _CHUNK


@functools.lru_cache(maxsize=None)
def _make_sc_gather():
    @functools.partial(
        pl.kernel,
        mesh=plsc.VectorSubcoreMesh(core_axis_name="c", subcore_axis_name="s"),
        out_type=jax.ShapeDtypeStruct((_B_SLICE, H // 2), jnp.int32),
        scratch_types=[
            pltpu.VMEM((_PER_W,), jnp.int32),
            pltpu.VMEM((_CHUNK, H // 2), jnp.int32),
            pltpu.VMEM((_CHUNK, H // 2), jnp.int32),
            pltpu.SemaphoreType.DMA,
            pltpu.SemaphoreType.DMA,
        ],
    )
    def _sc_gather(t_hbm, i_hbm, o_hbm, idx_v, rows0, rows1, sem0, sem1):
        wid = lax.axis_index("s") * _NC + lax.axis_index("c")
        base = wid * _PER_W
        pltpu.sync_copy(i_hbm.at[pl.ds(base, _PER_W)], idx_v)

        def g_start(ci, rows, sem):
            pltpu.make_async_copy(
                t_hbm.at[idx_v.at[pl.ds(ci * _CHUNK, _CHUNK)]], rows, sem
            ).start()

        def g_wait(ci, rows, sem):
            pltpu.make_async_copy(
                t_hbm.at[idx_v.at[pl.ds(ci * _CHUNK, _CHUNK)]], rows, sem
            ).wait()

        def wb(ci, rows):
            pltpu.sync_copy(rows, o_hbm.at[pl.ds(base + ci * _CHUNK, _CHUNK)])

        g_start(0, rows0, sem0)

        @pl.loop(0, (_NCHUNK - 1) // 2)
        def _(cp):
            ci0 = 2 * cp
            ci1 = ci0 + 1
            g_start(ci1, rows1, sem1)
            g_wait(ci0, rows0, sem0)
            wb(ci0, rows0)
            g_start(ci0 + 2, rows0, sem0)
            g_wait(ci1, rows1, sem1)
            wb(ci1, rows1)

        g_wait(_NCHUNK - 1, rows0, sem0)
        wb(_NCHUNK - 1, rows0)

    return _sc_gather


_FIN_BE = 2000
_FIN_NB = _E_S // _FIN_BE


def _unpack_lo(v):
    return jax.lax.bitcast_convert_type(v << 16, jnp.float32)


def _unpack_hi(v):
    return jax.lax.bitcast_convert_type(v & jnp.int32(-65536), jnp.float32)


def _finalize_kernel(s_ref, r_ref, ea_ref, g_ref, w1e_ref, w1g_ref, b1_ref,
                     w2_ref, b2_ref, *rest):
    o_ref = rest[-1]
    s = s_ref[...]
    r = r_ref[...]
    eap = jnp.dot(ea_ref[...].astype(_BF), w1e_ref[...].astype(_BF),
                  preferred_element_type=jnp.float32)
    gp = jnp.dot(g_ref[...].astype(_BF), w1g_ref[...].astype(_BF),
                 preferred_element_type=jnp.float32)
    base = eap + (gp + b1_ref[...])
    hm = H // 2
    h_lo = jnp.maximum(_unpack_lo(s) + _unpack_lo(r) + base[:, :hm], 0.0)
    h_hi = jnp.maximum(_unpack_hi(s) + _unpack_hi(r) + base[:, hm:], 0.0)
    o_ref[...] = (
        jnp.dot(h_lo.astype(_BF), w2_ref[:hm, :].astype(_BF),
                preferred_element_type=jnp.float32)
        + jnp.dot(h_hi.astype(_BF), w2_ref[hm:, :].astype(_BF),
                  preferred_element_type=jnp.float32)
        + b2_ref[...])


def _finalize_slice(k, gathered, ea, g, w1e, w1g, b1, w2, b2, prev=None):
    nb = _FIN_NB
    in_specs = [
        pl.BlockSpec((_FIN_BE, H // 2), lambda i: (i, 0)),
        pl.BlockSpec((_FIN_BE, H // 2), lambda i: (i + nb, 0)),
        pl.BlockSpec((_FIN_BE, DE), lambda i: (i + k * nb, 0)),
        pl.BlockSpec((1, DG), lambda i: (0, 0)),
        pl.BlockSpec((DE, H), lambda i: (0, 0)),
        pl.BlockSpec((DG, H), lambda i: (0, 0)),
        pl.BlockSpec((1, H), lambda i: (0, 0)),
        pl.BlockSpec((H, DOUT), lambda i: (0, 0)),
        pl.BlockSpec((1, DOUT), lambda i: (0, 0)),
    ]
    args = [gathered, gathered, ea, g, w1e, w1g, b1, w2, b2]
    io_aliases = {}
    if prev is not None:
        in_specs.append(pl.BlockSpec(memory_space=pltpu.MemorySpace.HBM))
        args.append(prev)
        io_aliases = {9: 0}
    return pl.pallas_call(
        _finalize_kernel,
        grid=(nb,),
        in_specs=in_specs,
        out_specs=pl.BlockSpec((_FIN_BE, DOUT), lambda i: (i + k * nb, 0)),
        out_shape=jax.ShapeDtypeStruct((E, DOUT), jnp.float32),
        input_output_aliases=io_aliases,
    )(*args)


def kernel(node_attributes, edge_index, edge_attributes, global_attributes,
           W1, b1, W2, b2):
    w1e = W1[:DE]
    w1s = W1[DE:DE + D]
    w1r = W1[DE + D:DE + 2 * D]
    w1g = W1[DE + 2 * D:]
    w_sr = jnp.concatenate([w1s, w1r], axis=1)

    table = _project_nodes(node_attributes, w_sr)

    src = edge_index[0]
    dst_n = edge_index[1] + N
    gathered = []
    for k in range(_NSLICE):
        a = k * _E_S
        idx_k = jnp.concatenate(
            [src[a:a + _E_S], dst_n[a:a + _E_S]]).astype(jnp.int32)
        gathered.append(_make_sc_gather()(table, idx_k))

    g2 = global_attributes.reshape(1, DG)
    b1r = b1.reshape(1, H)
    b2r = b2.reshape(1, DOUT)
    out = _finalize_slice(0, gathered[0], edge_attributes, g2, w1e, w1g,
                          b1r, W2, b2r)
    out = _finalize_slice(1, gathered[1], edge_attributes, g2, w1e, w1g,
                          b1r, W2, b2r, prev=out)
    return out


# speedup vs baseline: 1.2807x; 1.2807x over previous
import functools

import jax
import jax.numpy as jnp
from jax import lax
from jax.experimental import pallas as pl
from jax.experimental.pallas import tpu as pltpu
from jax.experimental.pallas import tpu_sc as plsc

N = 10000
E = 160000
D = 256
DE = 16
DG = 128
H = 512
DOUT = 256

_BF = jnp.bfloat16
_DH = D // 2


_PACK_BN = 2000


def _pack_kernel(x_ref, o_ref):
    bits = jax.lax.bitcast_convert_type(
        x_ref[...].astype(_BF).astype(jnp.float32), jnp.int32) >> 16
    lo = bits[:, :_DH] & jnp.int32(0xFFFF)
    hi = bits[:, _DH:] << 16
    o_ref[...] = hi | lo


def _pack_nodes(x):
    nb = N // _PACK_BN
    return pl.pallas_call(
        _pack_kernel,
        grid=(nb,),
        in_specs=[pl.BlockSpec((_PACK_BN, D), lambda i: (i, 0))],
        out_specs=pl.BlockSpec((_PACK_BN, _DH), lambda i: (i, 0)),
        out_shape=jax.ShapeDtypeStruct((N, _DH), jnp.int32),
    )(x)


_NC = 2
_NS = 16
_NW = _NC * _NS
_E_SLICES = (30720, 51200, 51200, 26880)
_CHUNK = 80


@functools.lru_cache(maxsize=None)
def _make_sc_gather(b_slice):
    per_w = b_slice // _NW
    nchunk = per_w // _CHUNK
    assert per_w % _CHUNK == 0 and nchunk >= 3

    @functools.partial(
        pl.kernel,
        mesh=plsc.VectorSubcoreMesh(core_axis_name="c", subcore_axis_name="s"),
        out_type=jax.ShapeDtypeStruct((b_slice, _DH), jnp.int32),
        scratch_types=[
            pltpu.VMEM((per_w,), jnp.int32),
            pltpu.VMEM((_CHUNK, _DH), jnp.int32),
            pltpu.VMEM((_CHUNK, _DH), jnp.int32),
            pltpu.SemaphoreType.DMA,
            pltpu.SemaphoreType.DMA,
        ],
    )
    def _sc_gather(t_hbm, i_hbm, o_hbm, idx_v, rows0, rows1, sem0, sem1):
        wid = lax.axis_index("s") * _NC + lax.axis_index("c")
        base = wid * per_w
        pltpu.sync_copy(i_hbm.at[pl.ds(base, per_w)], idx_v)

        def g_start(ci, rows, sem):
            pltpu.make_async_copy(
                t_hbm.at[idx_v.at[pl.ds(ci * _CHUNK, _CHUNK)]], rows, sem
            ).start()

        def g_wait(ci, rows, sem):
            pltpu.make_async_copy(
                t_hbm.at[idx_v.at[pl.ds(ci * _CHUNK, _CHUNK)]], rows, sem
            ).wait()

        def wb(ci, rows):
            pltpu.sync_copy(rows, o_hbm.at[pl.ds(base + ci * _CHUNK, _CHUNK)])

        g_start(0, rows0, sem0)
        g_start(1, rows1, sem1)

        npairs = (nchunk - 2) // 2

        @pl.loop(0, npairs)
        def _(cp):
            ci0 = 2 * cp
            ci1 = ci0 + 1
            g_wait(ci0, rows0, sem0)
            wb(ci0, rows0)
            g_start(ci0 + 2, rows0, sem0)
            g_wait(ci1, rows1, sem1)
            wb(ci1, rows1)
            g_start(ci1 + 2, rows1, sem1)

        b0 = 2 * npairs
        b1 = b0 + 1
        rem = nchunk - 2 - 2 * npairs
        g_wait(b0, rows0, sem0)
        wb(b0, rows0)
        if rem:
            g_start(nchunk - 1, rows0, sem0)
        g_wait(b1, rows1, sem1)
        wb(b1, rows1)
        if rem:
            g_wait(nchunk - 1, rows0, sem0)
            wb(nchunk - 1, rows0)

    return _sc_gather


_FIN_BE = 1280


def _unpack_lo(v):
    return jax.lax.bitcast_convert_type(v << 16, jnp.float32)


def _unpack_hi(v):
    return jax.lax.bitcast_convert_type(v & jnp.int32(-65536), jnp.float32)


def _unpack_attrs(v):
    return jnp.concatenate(
        [_unpack_lo(v).astype(_BF), _unpack_hi(v).astype(_BF)], axis=1)


def _finalize_kernel(s_ref, r_ref, ea_ref, g_ref, w1e_ref, w1g_ref, b1_ref,
                     w_sr_ref, w2_ref, b2_ref, *rest):
    o_ref = rest[-1]
    x = jnp.concatenate(
        [_unpack_attrs(s_ref[...]), _unpack_attrs(r_ref[...])], axis=1)
    node_t = jnp.dot(x, w_sr_ref[...].astype(_BF),
                     preferred_element_type=jnp.float32)
    eap = jax.lax.dot_general(
        ea_ref[0].astype(_BF), w1e_ref[...].astype(_BF),
        dimension_numbers=(((0,), (0,)), ((), ())),
        preferred_element_type=jnp.float32)
    gp = jnp.dot(g_ref[...].astype(_BF), w1g_ref[...].astype(_BF),
                 preferred_element_type=jnp.float32)
    h = jnp.maximum(node_t + eap + (gp + b1_ref[...]), 0.0)
    o_ref[...] = jnp.dot(h.astype(_BF), w2_ref[...].astype(_BF),
                         preferred_element_type=jnp.float32) + b2_ref[...]


def _finalize_slice(e_start, e_count, gathered, ea, g, w1e, w1g, b1, w_sr,
                    w2, b2, prev=None):
    nb = e_count // _FIN_BE
    off = e_start // _FIN_BE
    in_specs = [
        pl.BlockSpec((_FIN_BE, _DH), lambda i: (i, 0)),
        pl.BlockSpec((_FIN_BE, _DH), lambda i: (i + nb, 0)),
        pl.BlockSpec((1, DE, _FIN_BE), lambda i: (i + off, 0, 0)),
        pl.BlockSpec((1, DG), lambda i: (0, 0)),
        pl.BlockSpec((DE, H), lambda i: (0, 0)),
        pl.BlockSpec((DG, H), lambda i: (0, 0)),
        pl.BlockSpec((1, H), lambda i: (0, 0)),
        pl.BlockSpec((2 * D, H), lambda i: (0, 0)),
        pl.BlockSpec((H, DOUT), lambda i: (0, 0)),
        pl.BlockSpec((1, DOUT), lambda i: (0, 0)),
    ]
    args = [gathered, gathered, ea, g, w1e, w1g, b1, w_sr, w2, b2]
    io_aliases = {}
    if prev is not None:
        in_specs.append(pl.BlockSpec(memory_space=pltpu.MemorySpace.HBM))
        args.append(prev)
        io_aliases = {10: 0}
    return pl.pallas_call(
        _finalize_kernel,
        grid=(nb,),
        in_specs=in_specs,
        out_specs=pl.BlockSpec((_FIN_BE, DOUT), lambda i: (i + off, 0)),
        out_shape=jax.ShapeDtypeStruct((E, DOUT), jnp.float32),
        input_output_aliases=io_aliases,
    )(*args)


def kernel(node_attributes, edge_index, edge_attributes, global_attributes,
           W1, b1, W2, b2):
    w1e = W1[:DE]
    w_sr = W1[DE:DE + 2 * D]
    w1g = W1[DE + 2 * D:]

    table = _pack_nodes(node_attributes)

    src = edge_index[0]
    dst = edge_index[1]
    gathered = []
    starts = []
    a = 0
    for e_k in _E_SLICES:
        idx_k = jnp.concatenate(
            [src[a:a + e_k], dst[a:a + e_k]]).astype(jnp.int32)
        gathered.append(_make_sc_gather(2 * e_k)(table, idx_k))
        starts.append(a)
        a += e_k

    g2 = global_attributes.reshape(1, DG)
    b1r = b1.reshape(1, H)
    b2r = b2.reshape(1, DOUT)
    ea_t = edge_attributes.reshape(E // _FIN_BE, _FIN_BE, DE).transpose(0, 2, 1)
    out = None
    for e_start, e_k, g_k in zip(starts, _E_SLICES, gathered):
        out = _finalize_slice(e_start, e_k, g_k, ea_t, g2, w1e, w1g,
                              b1r, w_sr, W2, b2r, prev=out)
    return out


# speedup vs baseline: 1.3456x; 1.0507x over previous
import functools

import jax
import jax.numpy as jnp
from jax import lax
from jax.experimental import pallas as pl
from jax.experimental.pallas import tpu as pltpu
from jax.experimental.pallas import tpu_sc as plsc

N = 10000
E = 160000
D = 256
DE = 16
DG = 128
H = 512
DOUT = 256

_BF = jnp.bfloat16
_DH = D // 2


_PACK_BN = 2000


def _pack_kernel(x_ref, o_ref):
    bits = jax.lax.bitcast_convert_type(
        x_ref[...].astype(_BF).astype(jnp.float32), jnp.int32) >> 16
    lo = bits[:, :_DH] & jnp.int32(0xFFFF)
    hi = bits[:, _DH:] << 16
    o_ref[...] = hi | lo


def _pack_nodes(x):
    nb = N // _PACK_BN
    return pl.pallas_call(
        _pack_kernel,
        grid=(nb,),
        in_specs=[pl.BlockSpec((_PACK_BN, D), lambda i: (i, 0))],
        out_specs=pl.BlockSpec((_PACK_BN, _DH), lambda i: (i, 0)),
        out_shape=jax.ShapeDtypeStruct((N, _DH), jnp.int32),
    )(x)


_NC = 2
_NS = 16
_NW = _NC * _NS
_E_SLICES = (32000, 51200, 51200, 25600)
_CHUNK = 80


@functools.lru_cache(maxsize=None)
def _make_sc_gather(b_slice):
    per_w = b_slice // _NW
    nchunk = per_w // _CHUNK
    assert per_w % _CHUNK == 0 and nchunk >= 3

    @functools.partial(
        pl.kernel,
        mesh=plsc.VectorSubcoreMesh(core_axis_name="c", subcore_axis_name="s"),
        out_type=jax.ShapeDtypeStruct((b_slice, _DH), jnp.int32),
        scratch_types=[
            pltpu.VMEM((per_w,), jnp.int32),
            pltpu.VMEM((_CHUNK, _DH), jnp.int32),
            pltpu.VMEM((_CHUNK, _DH), jnp.int32),
            pltpu.SemaphoreType.DMA,
            pltpu.SemaphoreType.DMA,
        ],
    )
    def _sc_gather(t_hbm, i_hbm, o_hbm, idx_v, rows0, rows1, sem0, sem1):
        wid = lax.axis_index("s") * _NC + lax.axis_index("c")
        base = wid * per_w
        pltpu.sync_copy(i_hbm.at[pl.ds(base, per_w)], idx_v)

        def g_start(ci, rows, sem):
            pltpu.make_async_copy(
                t_hbm.at[idx_v.at[pl.ds(ci * _CHUNK, _CHUNK)]], rows, sem
            ).start()

        def g_wait(ci, rows, sem):
            pltpu.make_async_copy(
                t_hbm.at[idx_v.at[pl.ds(ci * _CHUNK, _CHUNK)]], rows, sem
            ).wait()

        def wb(ci, rows):
            pltpu.sync_copy(rows, o_hbm.at[pl.ds(base + ci * _CHUNK, _CHUNK)])

        g_start(0, rows0, sem0)
        g_start(1, rows1, sem1)

        npairs = (nchunk - 2) // 2

        @pl.loop(0, npairs)
        def _(cp):
            ci0 = 2 * cp
            ci1 = ci0 + 1
            g_wait(ci0, rows0, sem0)
            wb(ci0, rows0)
            g_start(ci0 + 2, rows0, sem0)
            g_wait(ci1, rows1, sem1)
            wb(ci1, rows1)
            g_start(ci1 + 2, rows1, sem1)

        b0 = 2 * npairs
        b1 = b0 + 1
        rem = nchunk - 2 - 2 * npairs
        g_wait(b0, rows0, sem0)
        wb(b0, rows0)
        if rem:
            g_start(nchunk - 1, rows0, sem0)
        g_wait(b1, rows1, sem1)
        wb(b1, rows1)
        if rem:
            g_wait(nchunk - 1, rows0, sem0)
            wb(nchunk - 1, rows0)

    return _sc_gather


_FIN_BE = 3200


def _unpack_lo(v):
    return jax.lax.bitcast_convert_type(v << 16, jnp.float32)


def _unpack_hi(v):
    return jax.lax.bitcast_convert_type(v & jnp.int32(-65536), jnp.float32)


def _unpack_attrs(v):
    return jnp.concatenate(
        [_unpack_lo(v).astype(_BF), _unpack_hi(v).astype(_BF)], axis=1)


def _finalize_kernel(s_ref, r_ref, ea_ref, g_ref, w1e_ref, w1g_ref, b1_ref,
                     w_sr_ref, w2_ref, b2_ref, *rest):
    o_ref = rest[-1]
    x = jnp.concatenate(
        [_unpack_attrs(s_ref[...]), _unpack_attrs(r_ref[...])], axis=1)
    node_t = jnp.dot(x, w_sr_ref[...].astype(_BF),
                     preferred_element_type=jnp.float32)
    eap = jax.lax.dot_general(
        ea_ref[0].astype(_BF), w1e_ref[...].astype(_BF),
        dimension_numbers=(((0,), (0,)), ((), ())),
        preferred_element_type=jnp.float32)
    gp = jnp.dot(g_ref[...].astype(_BF), w1g_ref[...].astype(_BF),
                 preferred_element_type=jnp.float32)
    h = jnp.maximum(node_t + eap + (gp + b1_ref[...]), 0.0)
    o_ref[...] = jnp.dot(h.astype(_BF), w2_ref[...].astype(_BF),
                         preferred_element_type=jnp.float32) + b2_ref[...]


def _finalize_slice(e_start, e_count, gathered, ea, g, w1e, w1g, b1, w_sr,
                    w2, b2, prev=None):
    nb = e_count // _FIN_BE
    off = e_start // _FIN_BE
    in_specs = [
        pl.BlockSpec((_FIN_BE, _DH), lambda i: (i, 0)),
        pl.BlockSpec((_FIN_BE, _DH), lambda i: (i + nb, 0)),
        pl.BlockSpec((1, DE, _FIN_BE), lambda i: (i + off, 0, 0)),
        pl.BlockSpec((1, DG), lambda i: (0, 0)),
        pl.BlockSpec((DE, H), lambda i: (0, 0)),
        pl.BlockSpec((DG, H), lambda i: (0, 0)),
        pl.BlockSpec((1, H), lambda i: (0, 0)),
        pl.BlockSpec((2 * D, H), lambda i: (0, 0)),
        pl.BlockSpec((H, DOUT), lambda i: (0, 0)),
        pl.BlockSpec((1, DOUT), lambda i: (0, 0)),
    ]
    args = [gathered, gathered, ea, g, w1e, w1g, b1, w_sr, w2, b2]
    io_aliases = {}
    if prev is not None:
        in_specs.append(pl.BlockSpec(memory_space=pltpu.MemorySpace.HBM))
        args.append(prev)
        io_aliases = {10: 0}
    return pl.pallas_call(
        _finalize_kernel,
        grid=(nb,),
        in_specs=in_specs,
        out_specs=pl.BlockSpec((_FIN_BE, DOUT), lambda i: (i + off, 0)),
        out_shape=jax.ShapeDtypeStruct((E, DOUT), jnp.float32),
        input_output_aliases=io_aliases,
    )(*args)


def kernel(node_attributes, edge_index, edge_attributes, global_attributes,
           W1, b1, W2, b2):
    w1e = W1[:DE]
    w_sr = W1[DE:DE + 2 * D]
    w1g = W1[DE + 2 * D:]

    table = _pack_nodes(node_attributes)

    src = edge_index[0]
    dst = edge_index[1]
    gathered = []
    starts = []
    a = 0
    for e_k in _E_SLICES:
        idx_k = jnp.concatenate(
            [src[a:a + e_k], dst[a:a + e_k]]).astype(jnp.int32)
        gathered.append(_make_sc_gather(2 * e_k)(table, idx_k))
        starts.append(a)
        a += e_k

    g2 = global_attributes.reshape(1, DG)
    b1r = b1.reshape(1, H)
    b2r = b2.reshape(1, DOUT)
    ea_t = edge_attributes.reshape(E // _FIN_BE, _FIN_BE, DE).transpose(0, 2, 1)
    out = None
    for e_start, e_k, g_k in zip(starts, _E_SLICES, gathered):
        out = _finalize_slice(e_start, e_k, g_k, ea_t, g2, w1e, w1g,
                              b1r, w_sr, W2, b2r, prev=out)
    return out


# speedup vs baseline: 1.3684x; 1.0170x over previous
import functools

import jax
import jax.numpy as jnp
from jax import lax
from jax.experimental import pallas as pl
from jax.experimental.pallas import tpu as pltpu
from jax.experimental.pallas import tpu_sc as plsc

N = 10000
E = 160000
D = 256
DE = 16
DG = 128
H = 512
DOUT = 256

_BF = jnp.bfloat16
_DH = D // 2


_PACK_BN = 2000


def _pack_kernel(x_ref, o_ref):
    bits = jax.lax.bitcast_convert_type(
        x_ref[...].astype(_BF).astype(jnp.float32), jnp.int32) >> 16
    lo = bits[:, :_DH] & jnp.int32(0xFFFF)
    hi = bits[:, _DH:] << 16
    o_ref[...] = hi | lo


def _pack_nodes(x):
    nb = N // _PACK_BN
    return pl.pallas_call(
        _pack_kernel,
        grid=(nb,),
        in_specs=[pl.BlockSpec((_PACK_BN, D), lambda i: (i, 0))],
        out_specs=pl.BlockSpec((_PACK_BN, _DH), lambda i: (i, 0)),
        out_shape=jax.ShapeDtypeStruct((N, _DH), jnp.int32),
    )(x)


_NC = 2
_NS = 16
_NW = _NC * _NS
_E_SLICES = (32000, 51200, 51200, 25600)
_CHUNK = 80


@functools.lru_cache(maxsize=None)
def _make_sc_gather(b_slice):
    per_w = b_slice // _NW
    nchunk = per_w // _CHUNK
    assert per_w % _CHUNK == 0 and nchunk >= 3

    @functools.partial(
        pl.kernel,
        mesh=plsc.VectorSubcoreMesh(core_axis_name="c", subcore_axis_name="s"),
        out_type=jax.ShapeDtypeStruct((b_slice, _DH), jnp.int32),
        scratch_types=[
            pltpu.VMEM((per_w,), jnp.int32),
            pltpu.VMEM((_CHUNK, _DH), jnp.int32),
            pltpu.VMEM((_CHUNK, _DH), jnp.int32),
            pltpu.SemaphoreType.DMA,
            pltpu.SemaphoreType.DMA,
        ],
    )
    def _sc_gather(t_hbm, i_hbm, o_hbm, idx_v, rows0, rows1, sem0, sem1):
        wid = lax.axis_index("s") * _NC + lax.axis_index("c")
        base = wid * per_w
        pltpu.sync_copy(i_hbm.at[pl.ds(base, per_w)], idx_v)

        def g_start(ci, rows, sem):
            pltpu.make_async_copy(
                t_hbm.at[idx_v.at[pl.ds(ci * _CHUNK, _CHUNK)]], rows, sem
            ).start()

        def g_wait(ci, rows, sem):
            pltpu.make_async_copy(
                t_hbm.at[idx_v.at[pl.ds(ci * _CHUNK, _CHUNK)]], rows, sem
            ).wait()

        def wb(ci, rows):
            pltpu.sync_copy(rows, o_hbm.at[pl.ds(base + ci * _CHUNK, _CHUNK)])

        g_start(0, rows0, sem0)
        g_start(1, rows1, sem1)

        npairs = (nchunk - 2) // 2

        @pl.loop(0, npairs)
        def _(cp):
            ci0 = 2 * cp
            ci1 = ci0 + 1
            g_wait(ci0, rows0, sem0)
            wb(ci0, rows0)
            g_start(ci0 + 2, rows0, sem0)
            g_wait(ci1, rows1, sem1)
            wb(ci1, rows1)
            g_start(ci1 + 2, rows1, sem1)

        b0 = 2 * npairs
        b1 = b0 + 1
        rem = nchunk - 2 - 2 * npairs
        g_wait(b0, rows0, sem0)
        wb(b0, rows0)
        if rem:
            g_start(nchunk - 1, rows0, sem0)
        g_wait(b1, rows1, sem1)
        wb(b1, rows1)
        if rem:
            g_wait(nchunk - 1, rows0, sem0)
            wb(nchunk - 1, rows0)

    return _sc_gather


_FIN_BE = 3200


def _unpack_lo(v):
    return jax.lax.bitcast_convert_type(v << 16, jnp.float32)


def _unpack_hi(v):
    return jax.lax.bitcast_convert_type(v & jnp.int32(-65536), jnp.float32)


def _unpack_attrs(v):
    return jnp.concatenate(
        [_unpack_lo(v).astype(_BF), _unpack_hi(v).astype(_BF)], axis=1)


def _finalize_kernel(s_ref, r_ref, ea_ref, g_ref, w1e_ref, w1g_ref, b1_ref,
                     w_sr_ref, w2_ref, b2_ref, *rest):
    o_ref = rest[-1]
    x = jnp.concatenate(
        [_unpack_attrs(s_ref[...]), _unpack_attrs(r_ref[...])], axis=1)
    node_t = jnp.dot(x, w_sr_ref[...].astype(_BF),
                     preferred_element_type=jnp.float32).astype(_BF)
    eap = jax.lax.dot_general(
        ea_ref[0].astype(_BF), w1e_ref[...].astype(_BF),
        dimension_numbers=(((0,), (0,)), ((), ())),
        preferred_element_type=jnp.float32).astype(_BF)
    gpb = (jnp.dot(g_ref[...].astype(_BF), w1g_ref[...].astype(_BF),
                   preferred_element_type=jnp.float32)
           + b1_ref[...]).astype(_BF)
    h = jnp.maximum(node_t + (eap + gpb), jnp.array(0.0, _BF))
    o_ref[...] = jnp.dot(h, w2_ref[...].astype(_BF),
                         preferred_element_type=jnp.float32) + b2_ref[...]


def _finalize_slice(e_start, e_count, gathered, ea, g, w1e, w1g, b1, w_sr,
                    w2, b2, prev=None):
    nb = e_count // _FIN_BE
    off = e_start // _FIN_BE
    in_specs = [
        pl.BlockSpec((_FIN_BE, _DH), lambda i: (i, 0)),
        pl.BlockSpec((_FIN_BE, _DH), lambda i: (i + nb, 0)),
        pl.BlockSpec((1, DE, _FIN_BE), lambda i: (i + off, 0, 0)),
        pl.BlockSpec((1, DG), lambda i: (0, 0)),
        pl.BlockSpec((DE, H), lambda i: (0, 0)),
        pl.BlockSpec((DG, H), lambda i: (0, 0)),
        pl.BlockSpec((1, H), lambda i: (0, 0)),
        pl.BlockSpec((2 * D, H), lambda i: (0, 0)),
        pl.BlockSpec((H, DOUT), lambda i: (0, 0)),
        pl.BlockSpec((1, DOUT), lambda i: (0, 0)),
    ]
    args = [gathered, gathered, ea, g, w1e, w1g, b1, w_sr, w2, b2]
    io_aliases = {}
    if prev is not None:
        in_specs.append(pl.BlockSpec(memory_space=pltpu.MemorySpace.HBM))
        args.append(prev)
        io_aliases = {10: 0}
    return pl.pallas_call(
        _finalize_kernel,
        grid=(nb,),
        in_specs=in_specs,
        out_specs=pl.BlockSpec((_FIN_BE, DOUT), lambda i: (i + off, 0)),
        out_shape=jax.ShapeDtypeStruct((E, DOUT), jnp.float32),
        input_output_aliases=io_aliases,
    )(*args)


def kernel(node_attributes, edge_index, edge_attributes, global_attributes,
           W1, b1, W2, b2):
    w1e = W1[:DE]
    w_sr = W1[DE:DE + 2 * D]
    w1g = W1[DE + 2 * D:]

    table = _pack_nodes(node_attributes)

    src = edge_index[0]
    dst = edge_index[1]
    gathered = []
    starts = []
    a = 0
    for e_k in _E_SLICES:
        idx_k = jnp.concatenate(
            [src[a:a + e_k], dst[a:a + e_k]]).astype(jnp.int32)
        gathered.append(_make_sc_gather(2 * e_k)(table, idx_k))
        starts.append(a)
        a += e_k

    g2 = global_attributes.reshape(1, DG)
    b1r = b1.reshape(1, H)
    b2r = b2.reshape(1, DOUT)
    ea_t = edge_attributes.reshape(E // _FIN_BE, _FIN_BE, DE).transpose(0, 2, 1)
    out = None
    for e_start, e_k, g_k in zip(starts, _E_SLICES, gathered):
        out = _finalize_slice(e_start, e_k, g_k, ea_t, g2, w1e, w1g,
                              b1r, w_sr, W2, b2r, prev=out)
    return out


# speedup vs baseline: 1.3836x; 1.0111x over previous
import functools

import jax
import jax.numpy as jnp
from jax import lax
from jax.experimental import pallas as pl
from jax.experimental.pallas import tpu as pltpu
from jax.experimental.pallas import tpu_sc as plsc

N = 10000
E = 160000
D = 256
DE = 16
DG = 128
H = 512
DOUT = 256

_BF = jnp.bfloat16
_DH = D // 2


_PACK_BN = 2000


def _pack_kernel(x_ref, o_ref):
    bits = jax.lax.bitcast_convert_type(
        x_ref[...].astype(_BF).astype(jnp.float32), jnp.int32) >> 16
    lo = bits[:, :_DH] & jnp.int32(0xFFFF)
    hi = bits[:, _DH:] << 16
    o_ref[...] = hi | lo


def _pack_nodes(x):
    nb = N // _PACK_BN
    return pl.pallas_call(
        _pack_kernel,
        grid=(nb,),
        in_specs=[pl.BlockSpec((_PACK_BN, D), lambda i: (i, 0))],
        out_specs=pl.BlockSpec((_PACK_BN, _DH), lambda i: (i, 0)),
        out_shape=jax.ShapeDtypeStruct((N, _DH), jnp.int32),
    )(x)


_NC = 2
_NS = 16
_NW = _NC * _NS
_E_SLICES = (32000, 51200, 51200, 25600)
_CHUNK = 80


@functools.lru_cache(maxsize=None)
def _make_sc_gather(b_slice):
    per_w = b_slice // _NW
    nchunk = per_w // _CHUNK
    assert per_w % _CHUNK == 0 and nchunk >= 3

    @functools.partial(
        pl.kernel,
        mesh=plsc.VectorSubcoreMesh(core_axis_name="c", subcore_axis_name="s"),
        out_type=jax.ShapeDtypeStruct((b_slice, _DH), jnp.int32),
        scratch_types=[
            pltpu.VMEM((per_w,), jnp.int32),
            pltpu.VMEM((_CHUNK, _DH), jnp.int32),
            pltpu.VMEM((_CHUNK, _DH), jnp.int32),
            pltpu.SemaphoreType.DMA,
            pltpu.SemaphoreType.DMA,
        ],
    )
    def _sc_gather(t_hbm, i_hbm, o_hbm, idx_v, rows0, rows1, sem0, sem1):
        wid = lax.axis_index("s") * _NC + lax.axis_index("c")
        base = wid * per_w
        pltpu.sync_copy(i_hbm.at[pl.ds(base, per_w)], idx_v)

        def g_start(ci, rows, sem):
            pltpu.make_async_copy(
                t_hbm.at[idx_v.at[pl.ds(ci * _CHUNK, _CHUNK)]], rows, sem
            ).start()

        def g_wait(ci, rows, sem):
            pltpu.make_async_copy(
                t_hbm.at[idx_v.at[pl.ds(ci * _CHUNK, _CHUNK)]], rows, sem
            ).wait()

        def wb(ci, rows):
            pltpu.sync_copy(rows, o_hbm.at[pl.ds(base + ci * _CHUNK, _CHUNK)])

        g_start(0, rows0, sem0)
        g_start(1, rows1, sem1)

        npairs = (nchunk - 2) // 2

        @pl.loop(0, npairs)
        def _(cp):
            ci0 = 2 * cp
            ci1 = ci0 + 1
            g_wait(ci0, rows0, sem0)
            wb(ci0, rows0)
            g_start(ci0 + 2, rows0, sem0)
            g_wait(ci1, rows1, sem1)
            wb(ci1, rows1)
            g_start(ci1 + 2, rows1, sem1)

        b0 = 2 * npairs
        b1 = b0 + 1
        rem = nchunk - 2 - 2 * npairs
        g_wait(b0, rows0, sem0)
        wb(b0, rows0)
        if rem:
            g_start(nchunk - 1, rows0, sem0)
        g_wait(b1, rows1, sem1)
        wb(b1, rows1)
        if rem:
            g_wait(nchunk - 1, rows0, sem0)
            wb(nchunk - 1, rows0)

    return _sc_gather


_FIN_BE = 6400


def _unpack_lo(v):
    return jax.lax.bitcast_convert_type(v << 16, jnp.float32)


def _unpack_hi(v):
    return jax.lax.bitcast_convert_type(v & jnp.int32(-65536), jnp.float32)


def _unpack_attrs(v):
    return jnp.concatenate(
        [_unpack_lo(v).astype(_BF), _unpack_hi(v).astype(_BF)], axis=1)


def _finalize_kernel(s_ref, r_ref, ea_ref, g_ref, w1e_ref, w1g_ref, b1_ref,
                     w_sr_ref, w2_ref, b2_ref, *rest):
    o_ref = rest[-1]
    x = jnp.concatenate(
        [_unpack_attrs(s_ref[...]), _unpack_attrs(r_ref[...])], axis=1)
    node_t = jnp.dot(x, w_sr_ref[...].astype(_BF),
                     preferred_element_type=jnp.float32).astype(_BF)
    eap = jax.lax.dot_general(
        ea_ref[0].astype(_BF), w1e_ref[...].astype(_BF),
        dimension_numbers=(((0,), (0,)), ((), ())),
        preferred_element_type=jnp.float32).astype(_BF)
    gpb = (jnp.dot(g_ref[...].astype(_BF), w1g_ref[...].astype(_BF),
                   preferred_element_type=jnp.float32)
           + b1_ref[...]).astype(_BF)
    h = jnp.maximum(node_t + (eap + gpb), jnp.array(0.0, _BF))
    o_ref[...] = jnp.dot(h, w2_ref[...].astype(_BF),
                         preferred_element_type=jnp.float32) + b2_ref[...]


def _finalize_slice(e_start, e_count, gathered, ea, g, w1e, w1g, b1, w_sr,
                    w2, b2, prev=None):
    nb = e_count // _FIN_BE
    off = e_start // _FIN_BE
    in_specs = [
        pl.BlockSpec((_FIN_BE, _DH), lambda i: (i, 0)),
        pl.BlockSpec((_FIN_BE, _DH), lambda i: (i + nb, 0)),
        pl.BlockSpec((1, DE, _FIN_BE), lambda i: (i + off, 0, 0)),
        pl.BlockSpec((1, DG), lambda i: (0, 0)),
        pl.BlockSpec((DE, H), lambda i: (0, 0)),
        pl.BlockSpec((DG, H), lambda i: (0, 0)),
        pl.BlockSpec((1, H), lambda i: (0, 0)),
        pl.BlockSpec((2 * D, H), lambda i: (0, 0)),
        pl.BlockSpec((H, DOUT), lambda i: (0, 0)),
        pl.BlockSpec((1, DOUT), lambda i: (0, 0)),
    ]
    args = [gathered, gathered, ea, g, w1e, w1g, b1, w_sr, w2, b2]
    io_aliases = {}
    if prev is not None:
        in_specs.append(pl.BlockSpec(memory_space=pltpu.MemorySpace.HBM))
        args.append(prev)
        io_aliases = {10: 0}
    return pl.pallas_call(
        _finalize_kernel,
        grid=(nb,),
        in_specs=in_specs,
        out_specs=pl.BlockSpec((_FIN_BE, DOUT), lambda i: (i + off, 0)),
        out_shape=jax.ShapeDtypeStruct((E, DOUT), jnp.float32),
        input_output_aliases=io_aliases,
    )(*args)


def kernel(node_attributes, edge_index, edge_attributes, global_attributes,
           W1, b1, W2, b2):
    w1e = W1[:DE]
    w_sr = W1[DE:DE + 2 * D]
    w1g = W1[DE + 2 * D:]

    table = _pack_nodes(node_attributes)

    src = edge_index[0]
    dst = edge_index[1]
    gathered = []
    starts = []
    a = 0
    for e_k in _E_SLICES:
        idx_k = jnp.concatenate(
            [src[a:a + e_k], dst[a:a + e_k]]).astype(jnp.int32)
        gathered.append(_make_sc_gather(2 * e_k)(table, idx_k))
        starts.append(a)
        a += e_k

    g2 = global_attributes.reshape(1, DG)
    b1r = b1.reshape(1, H)
    b2r = b2.reshape(1, DOUT)
    ea_t = edge_attributes.reshape(E // _FIN_BE, _FIN_BE, DE).transpose(0, 2, 1)
    out = None
    for e_start, e_k, g_k in zip(starts, _E_SLICES, gathered):
        out = _finalize_slice(e_start, e_k, g_k, ea_t, g2, w1e, w1g,
                              b1r, w_sr, W2, b2r, prev=out)
    return out
